# Initial kernel scaffold; baseline (speedup 1.0000x reference)
import jax
import jax.numpy as jnp
from jax.experimental import pallas as pl


def kernel(d_index, p_index, d_vecs, p_embeddings, y, d_ecfps, d_edge_index, d_edge_weight, p_gos, p_edge_index, p_edge_weight, Wd, bd, Wp, bp, We1, be1, We2, be2, Wdec1, bdec1, Wdec2, bdec2, Wo1, bo1, gamma, beta, Wo2, bo2):
    raise NotImplementedError("write your pallas kernel here")


# speedup vs baseline: 1.0914x; 1.0914x over previous
import functools

import jax
import jax.numpy as jnp
from jax.experimental import pallas as pl
from jax.experimental.pallas import tpu as pltpu

_INTERPRET = False


def _leaky(x):
    return jnp.where(x >= 0, x, 0.01 * x)


def _mm_body(x_ref, w_ref, o_ref):
    o_ref[...] = jnp.dot(x_ref[...], w_ref[...],
                         preferred_element_type=jnp.float32)


def _matmul(x, W, bm):
    M, K = x.shape
    N = W.shape[1]
    return pl.pallas_call(
        _mm_body,
        grid=(pl.cdiv(M, bm),),
        in_specs=[pl.BlockSpec((bm, K), lambda i: (i, 0)),
                  pl.BlockSpec((K, N), lambda i: (0, 0))],
        out_specs=pl.BlockSpec((bm, N), lambda i: (i, 0)),
        out_shape=jax.ShapeDtypeStruct((M, N), jnp.float32),
        interpret=_INTERPRET,
    )(x, W)


def _mlp_body(dv_ref, pe_ref, ec_ref, go_ref,
              We1_ref, be1_ref, We2_ref, be2_ref,
              Wd1_ref, bd1_ref, Wd2_ref, bd2_ref,
              Wo1_ref, bo1_ref,
              feat_ref, enc_ref, dec_ref, h_ref, stats_ref,
              acc_ref):
    feat = jnp.concatenate(
        [dv_ref[...], pe_ref[...], ec_ref[...], go_ref[...]], axis=1)
    feat_ref[...] = feat
    e1 = _leaky(jnp.dot(feat, We1_ref[...],
                        preferred_element_type=jnp.float32) + be1_ref[...])
    enc = _leaky(jnp.dot(e1, We2_ref[...],
                         preferred_element_type=jnp.float32) + be2_ref[...])
    enc_ref[...] = enc
    d1 = _leaky(jnp.dot(enc, Wd1_ref[...],
                        preferred_element_type=jnp.float32) + bd1_ref[...])
    dec_ref[...] = _leaky(jnp.dot(d1, Wd2_ref[...],
                                  preferred_element_type=jnp.float32) + bd2_ref[...])
    h = jnp.dot(enc, Wo1_ref[...],
                preferred_element_type=jnp.float32) + bo1_ref[...]
    h_ref[...] = h

    i = pl.program_id(0)

    @pl.when(i == 0)
    def _init():
        acc_ref[...] = jnp.zeros_like(acc_ref)

    acc_ref[0, :] += jnp.sum(h, axis=0)
    acc_ref[1, :] += jnp.sum(h * h, axis=0)

    @pl.when(i == pl.num_programs(0) - 1)
    def _emit():
        stats_ref[...] = acc_ref[...]


def _mlp(d_vecs, p_embeddings, ecfps_g, gos_g,
         We1, be1, We2, be2, Wdec1, bdec1, Wdec2, bdec2, Wo1, bo1, bm):
    Bn = d_vecs.shape[0]
    F0 = d_vecs.shape[1]
    F1 = p_embeddings.shape[1]
    F2 = ecfps_g.shape[1]
    F3 = gos_g.shape[1]
    FEAT = F0 + F1 + F2 + F3
    H1 = We1.shape[1]
    H2 = We2.shape[1]
    D1 = Wdec1.shape[1]
    D2 = Wdec2.shape[1]
    HO = Wo1.shape[1]
    grid = (Bn // bm,)

    def row_block(i):
        return (i, 0)

    def const_block(i):
        return (0, 0)

    def vec_block(i):
        return (0,)

    out_shapes = (
        jax.ShapeDtypeStruct((Bn, FEAT), jnp.float32),
        jax.ShapeDtypeStruct((Bn, H2), jnp.float32),
        jax.ShapeDtypeStruct((Bn, D2), jnp.float32),
        jax.ShapeDtypeStruct((Bn, HO), jnp.float32),
        jax.ShapeDtypeStruct((2, HO), jnp.float32),
    )
    out_specs = (
        pl.BlockSpec((bm, FEAT), row_block),
        pl.BlockSpec((bm, H2), row_block),
        pl.BlockSpec((bm, D2), row_block),
        pl.BlockSpec((bm, HO), row_block),
        pl.BlockSpec((2, HO), const_block),
    )
    in_specs = [
        pl.BlockSpec((bm, F0), row_block),
        pl.BlockSpec((bm, F1), row_block),
        pl.BlockSpec((bm, F2), row_block),
        pl.BlockSpec((bm, F3), row_block),
        pl.BlockSpec((FEAT, H1), const_block),
        pl.BlockSpec((H1,), vec_block),
        pl.BlockSpec((H1, H2), const_block),
        pl.BlockSpec((H2,), vec_block),
        pl.BlockSpec((H2, D1), const_block),
        pl.BlockSpec((D1,), vec_block),
        pl.BlockSpec((D1, D2), const_block),
        pl.BlockSpec((D2,), vec_block),
        pl.BlockSpec((H2, HO), const_block),
        pl.BlockSpec((HO,), vec_block),
    ]
    return pl.pallas_call(
        _mlp_body,
        grid=grid,
        in_specs=in_specs,
        out_specs=out_specs,
        out_shape=out_shapes,
        scratch_shapes=[pltpu.VMEM((2, HO), jnp.float32)],
        interpret=_INTERPRET,
    )(d_vecs, p_embeddings, ecfps_g, gos_g,
      We1, be1, We2, be2, Wdec1, bdec1, Wdec2, bdec2, Wo1, bo1)


def _head_body(h_ref, stats_ref, gamma_ref, beta_ref, Wo2_ref, bo2_ref,
               y_ref, *, inv_b):
    mean = stats_ref[0, :] * inv_b
    var = stats_ref[1, :] * inv_b - mean * mean
    hn = (h_ref[...] - mean) * jax.lax.rsqrt(var + 1e-5) * gamma_ref[...] \
        + beta_ref[...]
    hn = _leaky(hn)
    y_ref[...] = jnp.dot(hn, Wo2_ref[...],
                         preferred_element_type=jnp.float32) + bo2_ref[...]


def _head(h, stats, gamma, beta, Wo2, bo2, bm):
    Bn, HO = h.shape
    grid = (Bn // bm,)
    return pl.pallas_call(
        functools.partial(_head_body, inv_b=1.0 / Bn),
        grid=grid,
        in_specs=[
            pl.BlockSpec((bm, HO), lambda i: (i, 0)),
            pl.BlockSpec((2, HO), lambda i: (0, 0)),
            pl.BlockSpec((HO,), lambda i: (0,)),
            pl.BlockSpec((HO,), lambda i: (0,)),
            pl.BlockSpec((HO, 1), lambda i: (0, 0)),
            pl.BlockSpec((1,), lambda i: (0,)),
        ],
        out_specs=pl.BlockSpec((bm, 1), lambda i: (i, 0)),
        out_shape=jax.ShapeDtypeStruct((Bn, 1), jnp.float32),
        interpret=_INTERPRET,
    )(h, stats, gamma, beta, Wo2, bo2)


def _gcn_aggregate(h, edge_index, edge_weight, b):
    n = h.shape[0]
    src = edge_index[0]
    dst = edge_index[1]
    deg = jnp.zeros((n,), jnp.float32).at[dst].add(edge_weight) + 1.0
    dinv = jax.lax.rsqrt(deg)
    coeff = dinv[src] * edge_weight * dinv[dst]
    out = (h * (dinv * dinv)[:, None]).at[dst].add(coeff[:, None] * h[src])
    return _leaky(out + b)


def kernel(d_index, p_index, d_vecs, p_embeddings, y,
           d_ecfps, d_edge_index, d_edge_weight,
           p_gos, p_edge_index, p_edge_weight,
           Wd, bd, Wp, bp,
           We1, be1, We2, be2,
           Wdec1, bdec1, Wdec2, bdec2,
           Wo1, bo1, gamma, beta, Wo2, bo2):
    h_d = _matmul(d_ecfps, Wd, bm=400)
    h_p = _matmul(p_gos, Wp, bm=400)

    out_d = _gcn_aggregate(h_d, d_edge_index, d_edge_weight, bd)
    out_p = _gcn_aggregate(h_p, p_edge_index, p_edge_weight, bp)
    ecfps_g = out_d[d_index]
    gos_g = out_p[p_index]

    feature, encoded, decoded, h, stats = _mlp(
        d_vecs, p_embeddings, ecfps_g, gos_g,
        We1, be1, We2, be2, Wdec1, bdec1, Wdec2, bdec2, Wo1, bo1,
        bm=min(256, d_vecs.shape[0]))
    y_out = _head(h, stats, gamma, beta, Wo2, bo2,
                  bm=min(512, d_vecs.shape[0]))
    return (y_out, encoded, decoded, feature)
